# baseline (speedup 1.0000x reference)
import functools

import jax
import jax.numpy as jnp
from jax import lax
from jax.experimental import pallas as pl
from jax.experimental.pallas import tpu as pltpu
from jax.experimental.pallas import tpu_sc as plsc

EMBED_DIM = 64
NUM_CORES = 2
NUM_SUBCORES = 16
NUM_WORKERS = NUM_CORES * NUM_SUBCORES

GROUP = 256
RING = 5


def _make_gather(batch_total: int):
    b_per_w = batch_total // NUM_WORKERS
    num_groups = b_per_w // GROUP
    num_blocks = num_groups // RING

    mesh = plsc.VectorSubcoreMesh(core_axis_name="c", subcore_axis_name="s")

    @functools.partial(
        pl.kernel,
        mesh=mesh,
        out_type=jax.ShapeDtypeStruct((batch_total, EMBED_DIM), jnp.float32),
        compiler_params=pltpu.CompilerParams(use_tc_tiling_on_sc=False),
        scratch_types=(
            [pltpu.VMEM((b_per_w,), jnp.int32)]
            + [pltpu.VMEM((GROUP, EMBED_DIM), jnp.float32) for _ in range(RING)]
            + [pltpu.SemaphoreType.DMA for _ in range(2 * RING)]
        ),
    )
    def gather_kernel(idx_hbm, table_hbm, out_hbm, idx_v, *rest):
        rows = rest[:RING]
        gsem = rest[RING:2 * RING]
        ssem = rest[2 * RING:]

        wid = lax.axis_index("s") * NUM_CORES + lax.axis_index("c")
        base = wid * b_per_w

        pltpu.sync_copy(idx_hbm.at[pl.ds(base, b_per_w)], idx_v)

        def g_start(g, r):
            off = pl.multiple_of(g * GROUP, GROUP)
            pltpu.async_copy(table_hbm.at[idx_v.at[pl.ds(off, GROUP)]],
                             rows[r], gsem[r])

        def g_wait(g, r):
            off = pl.multiple_of(g * GROUP, GROUP)
            pltpu.make_async_copy(table_hbm.at[idx_v.at[pl.ds(off, GROUP)]],
                                  rows[r], gsem[r]).wait()

        def s_start(g, r):
            off = pl.multiple_of(base + g * GROUP, GROUP)
            pltpu.async_copy(rows[r], out_hbm.at[pl.ds(off, GROUP)], ssem[r])

        def s_wait(g, r):
            off = pl.multiple_of(base + g * GROUP, GROUP)
            pltpu.make_async_copy(rows[r], out_hbm.at[pl.ds(off, GROUP)],
                                  ssem[r]).wait()

        for r in range(RING):
            g_start(r, r)

        def body(blk, _):
            for r in range(RING):
                g = blk * RING + r
                g_wait(g, r)
                s_start(g, r)
            for r in range(RING):
                g = blk * RING + r
                s_wait(g, r)
                g_start(g + RING, r)
            return 0

        lax.fori_loop(0, num_blocks - 1, body, 0)

        last = (num_blocks - 1) * RING
        for r in range(RING):
            g_wait(last + r, r)
            s_start(last + r, r)
        for r in range(RING):
            s_wait(last + r, r)

    return gather_kernel


def kernel(idx_texts, embed_table):
    batch, seq = idx_texts.shape
    flat_idx = idx_texts.reshape(-1)
    out = _make_gather(batch * seq)(flat_idx, embed_table)
    return out.reshape(batch, seq, EMBED_DIM)


# speedup vs baseline: 1.0007x; 1.0007x over previous
import functools

import jax
import jax.numpy as jnp
from jax import lax
from jax.experimental import pallas as pl
from jax.experimental.pallas import tpu as pltpu
from jax.experimental.pallas import tpu_sc as plsc

EMBED_DIM = 64
NUM_CORES = 2
NUM_SUBCORES = 16
NUM_WORKERS = NUM_CORES * NUM_SUBCORES

RING = 4


def _make_gather(batch: int, seq: int):
    rows_per_w = batch // NUM_WORKERS
    num_blocks = rows_per_w // RING

    mesh = plsc.VectorSubcoreMesh(core_axis_name="c", subcore_axis_name="s")

    @functools.partial(
        pl.kernel,
        mesh=mesh,
        out_type=jax.ShapeDtypeStruct((batch, seq, EMBED_DIM), jnp.float32),
        compiler_params=pltpu.CompilerParams(use_tc_tiling_on_sc=False),
        scratch_types=(
            [pltpu.VMEM((rows_per_w, seq), jnp.int32)]
            + [pltpu.VMEM((seq, EMBED_DIM), jnp.float32) for _ in range(RING)]
            + [pltpu.SemaphoreType.DMA for _ in range(2 * RING)]
        ),
    )
    def gather_kernel(idx_hbm, table_hbm, out_hbm, idx_v, *rest):
        rows = rest[:RING]
        gsem = rest[RING:2 * RING]
        ssem = rest[2 * RING:]

        wid = lax.axis_index("s") * NUM_CORES + lax.axis_index("c")
        base = wid * rows_per_w

        pltpu.sync_copy(idx_hbm.at[pl.ds(base, rows_per_w)], idx_v)

        def g_copy(i, r):
            return pltpu.make_async_copy(
                table_hbm.at[idx_v.at[i]], rows[r], gsem[r])

        def s_copy(i, r):
            return pltpu.make_async_copy(
                rows[r], out_hbm.at[base + i], ssem[r])

        for r in range(RING):
            g_copy(r, r).start()

        def body(blk, _):
            for r in range(RING):
                i = blk * RING + r
                g_copy(i, r).wait()
                s_copy(i, r).start()
            for r in range(RING):
                i = blk * RING + r
                s_copy(i, r).wait()
                g_copy(i + RING, r).start()
            return 0

        lax.fori_loop(0, num_blocks - 1, body, 0)

        last = (num_blocks - 1) * RING
        for r in range(RING):
            g_copy(last + r, r).wait()
            s_copy(last + r, r).start()
        for r in range(RING):
            s_copy(last + r, r).wait()

    return gather_kernel


def kernel(idx_texts, embed_table):
    batch, seq = idx_texts.shape
    return _make_gather(batch, seq)(idx_texts, embed_table)


# speedup vs baseline: 1.3309x; 1.3300x over previous
import functools

import jax
import jax.numpy as jnp
from jax import lax
from jax.experimental import pallas as pl
from jax.experimental.pallas import tpu as pltpu
from jax.experimental.pallas import tpu_sc as plsc

EMBED_DIM = 64
NUM_CORES = 2
NUM_SUBCORES = 16
NUM_WORKERS = NUM_CORES * NUM_SUBCORES

RING = 4


def _make_gather(batch: int, seq: int):
    rows_per_w = batch // NUM_WORKERS
    num_blocks = rows_per_w // RING

    mesh = plsc.VectorSubcoreMesh(core_axis_name="c", subcore_axis_name="s")

    @functools.partial(
        pl.kernel,
        mesh=mesh,
        out_type=jax.ShapeDtypeStruct((batch, seq, 2 * EMBED_DIM), jnp.float32),
        compiler_params=pltpu.CompilerParams(use_tc_tiling_on_sc=False),
        scratch_types=(
            [pltpu.VMEM((rows_per_w, seq), jnp.int32)]
            + [pltpu.VMEM((seq, EMBED_DIM), jnp.float32) for _ in range(RING)]
            + [pltpu.SemaphoreType.DMA for _ in range(2 * RING)]
        ),
    )
    def gather_kernel(idx_hbm, table_hbm, out_hbm, idx_v, *rest):
        rows = rest[:RING]
        gsem = rest[RING:2 * RING]
        ssem = rest[2 * RING:]

        wid = lax.axis_index("s") * NUM_CORES + lax.axis_index("c")
        base = wid * rows_per_w

        pltpu.sync_copy(idx_hbm.at[pl.ds(base, rows_per_w)], idx_v)

        def g_copy(i, r):
            return pltpu.make_async_copy(
                table_hbm.at[idx_v.at[i]], rows[r], gsem[r])

        def s_copy(i, r):
            return pltpu.make_async_copy(
                rows[r], out_hbm.at[base + i, :, pl.ds(0, EMBED_DIM)], ssem[r])

        for r in range(RING):
            g_copy(r, r).start()

        def body(blk, _):
            for r in range(RING):
                i = blk * RING + r
                g_copy(i, r).wait()
                s_copy(i, r).start()
            for r in range(RING):
                i = blk * RING + r
                s_copy(i, r).wait()
                g_copy(i + RING, r).start()
            return 0

        lax.fori_loop(0, num_blocks - 1, body, 0)

        last = (num_blocks - 1) * RING
        for r in range(RING):
            g_copy(last + r, r).wait()
            s_copy(last + r, r).start()
        for r in range(RING):
            s_copy(last + r, r).wait()

    return gather_kernel


def kernel(idx_texts, embed_table):
    batch, seq = idx_texts.shape
    padded = _make_gather(batch, seq)(idx_texts, embed_table)
    return padded[:, :, :EMBED_DIM]
